# baseline (speedup 1.0000x reference)
import functools

import jax
import jax.numpy as jnp
from jax import lax
from jax.experimental import pallas as pl
from jax.experimental.pallas import tpu as pltpu
from jax.experimental.pallas import tpu_sc as plsc

VOCAB = 1000
EMBED_DIM = 128
BATCH = 4096
HIST = 200

_INFO = plsc.get_sparse_core_info()
NC = _INFO.num_cores
NS = _INFO.num_subcores
NW = NC * NS
TOTAL = BATCH * HIST
CHUNK = 128
PER_W = TOTAL // NW
NCHUNK = PER_W // CHUNK
NPAIR = NCHUNK // 2
NBUF = 2

_mesh = plsc.VectorSubcoreMesh(core_axis_name="c", subcore_axis_name="s")


@functools.partial(
    pl.kernel,
    mesh=_mesh,
    out_type=jax.ShapeDtypeStruct((TOTAL // CHUNK, CHUNK, EMBED_DIM),
                                  jnp.float32),
    scratch_types=[
        pltpu.VMEM((NCHUNK, CHUNK), jnp.int32),
        pltpu.VMEM((NBUF, 2, CHUNK, EMBED_DIM), jnp.float32),
        pltpu.VMEM_SHARED((VOCAB, EMBED_DIM), jnp.float32),
        pltpu.SemaphoreType.DMA((NBUF,)),
        pltpu.SemaphoreType.DMA((NBUF,)),
    ],
)
def _gather_kernel(idx_hbm, table_hbm, out_hbm, idx_v, rows_v, table_sh,
                   gsem, ssem):
    sid = lax.axis_index("s")
    wid = sid * NC + lax.axis_index("c")
    base = wid * NCHUNK

    @pl.when(sid == 0)
    def _stage():
        pltpu.sync_copy(table_hbm, table_sh)

    pltpu.sync_copy(idx_hbm.at[wid], idx_v)
    plsc.subcore_barrier()

    def _gather_half(jp, b, h):
        return pltpu.make_async_copy(
            table_sh.at[idx_v.at[2 * jp + h]], rows_v.at[b].at[h], gsem.at[b])

    class gather:
        def __init__(self, jp, b):
            self.halves = [_gather_half(jp, b, h) for h in range(2)]

        def start(self):
            for h in self.halves:
                h.start()

        def wait(self):
            for h in self.halves:
                h.wait()

    def store(jp, b):
        return pltpu.make_async_copy(
            rows_v.at[b], out_hbm.at[pl.ds(base + 2 * jp, 2)], ssem.at[b])

    gather(0, 0).start()

    def step(i, carry):
        for u in range(NBUF):
            jp = i * NBUF + u
            gather(jp, u).wait()
            store(jp, u).start()
            bn = 1 - u
            if u == 0:
                @pl.when(i > 0)
                def _wait_prev():
                    store(jp - 1, bn).wait()
                gather(jp + 1, bn).start()
            else:
                store(jp - 1, bn).wait()

                @pl.when(jp + 1 < NPAIR)
                def _prefetch():
                    gather(jp + 1, bn).start()
        return carry

    lax.fori_loop(0, NPAIR // NBUF, step, 0)

    store(NPAIR - 1, 1).wait()


def kernel(batch, weight):
    idx = batch.astype(jnp.int32).reshape(NW, NCHUNK, CHUNK)
    out = _gather_kernel(idx, weight)
    return out.reshape(BATCH, HIST, EMBED_DIM)
